# Initial kernel scaffold; baseline (speedup 1.0000x reference)
import jax
import jax.numpy as jnp
from jax.experimental import pallas as pl


def kernel(x, pe):
    raise NotImplementedError("write your pallas kernel here")


# speedup vs baseline: 1.9757x; 1.9757x over previous
import functools

import jax
import jax.numpy as jnp
from jax import lax
from jax.experimental import pallas as pl
from jax.experimental.pallas import tpu as pltpu
from jax.experimental.pallas import tpu_sc as plsc

_NC = 2
_NS = 16
_NW = _NC * _NS


@functools.lru_cache(maxsize=None)
def _make_gather(n, v, d, c):
    n_per_w = n // _NW
    n_chunks = n_per_w // c
    mesh = plsc.VectorSubcoreMesh(core_axis_name="c", subcore_axis_name="s")

    @functools.partial(
        pl.kernel,
        mesh=mesh,
        out_type=jax.ShapeDtypeStruct((n, d), jnp.float32),
        scratch_types=[
            pltpu.VMEM((n_chunks, c), jnp.int32),
            pltpu.VMEM((c, d), jnp.float32),
            pltpu.SemaphoreType.DMA,
        ],
    )
    def k(idx_hbm, table_hbm, out_hbm, idx_v, rows_v, gsem):
        wid = lax.axis_index("s") * _NC + lax.axis_index("c")
        base = wid * n_per_w
        pltpu.sync_copy(idx_hbm.at[wid], idx_v)

        def body(j, carry):
            pltpu.async_copy(table_hbm.at[idx_v.at[j]], rows_v, gsem).wait()
            pltpu.sync_copy(rows_v, out_hbm.at[pl.ds(base + j * c, c)])
            return carry

        lax.fori_loop(0, n_chunks, body, 0)

    return k


def kernel(x, pe):
    b, s = x.shape
    v, d = pe.shape
    n = b * s
    c = 32
    idx = x.reshape(_NW, (n // _NW) // c, c).astype(jnp.int32)
    out = _make_gather(n, v, d, c)(idx, pe)
    return out.reshape(b, s, d)


# speedup vs baseline: 2.2963x; 1.1622x over previous
import functools

import jax
import jax.numpy as jnp
from jax import lax
from jax.experimental import pallas as pl
from jax.experimental.pallas import tpu as pltpu
from jax.experimental.pallas import tpu_sc as plsc

_NC = 2
_NS = 16
_NW = _NC * _NS


@functools.lru_cache(maxsize=None)
def _make_gather(n, v, d, c):
    n_per_w = n // _NW
    n_chunks = n_per_w // c
    assert n_chunks >= 4 and n_chunks % 2 == 0
    mesh = plsc.VectorSubcoreMesh(core_axis_name="c", subcore_axis_name="s")

    @functools.partial(
        pl.kernel,
        mesh=mesh,
        out_type=jax.ShapeDtypeStruct((n, d), jnp.float32),
        scratch_types=[
            pltpu.VMEM((n_chunks, c), jnp.int32),
            pltpu.VMEM((2, c, d), jnp.float32),
            pltpu.SemaphoreType.DMA,
            pltpu.SemaphoreType.DMA,
            pltpu.SemaphoreType.DMA,
            pltpu.SemaphoreType.DMA,
        ],
    )
    def k(idx_hbm, table_hbm, out_hbm, idx_v, rows_v, g0, g1, w0, w1):
        wid = lax.axis_index("s") * _NC + lax.axis_index("c")
        base = wid * n_per_w
        gsems = (g0, g1)
        wsems = (w0, w1)
        pltpu.sync_copy(idx_hbm.at[wid], idx_v)

        def gath(j, b):
            return pltpu.make_async_copy(
                table_hbm.at[idx_v.at[j]], rows_v.at[b], gsems[b])

        def wr(j, b):
            return pltpu.make_async_copy(
                rows_v.at[b], out_hbm.at[pl.ds(base + j * c, c)], wsems[b])

        gath(0, 0).start()
        gath(0, 0).wait()
        wr(0, 0).start()
        gath(1, 1).start()

        def step(j, b):
            gath(j, b).wait()
            wr(j, b).start()
            wr(j - 1, 1 - b).wait()
            gath(j + 1, 1 - b).start()

        def body(i, carry):
            step(2 * i + 1, 1)
            step(2 * i + 2, 0)
            return carry

        lax.fori_loop(0, (n_chunks - 2) // 2, body, 0)
        j_last = n_chunks - 1
        gath(j_last, 1).wait()
        wr(j_last, 1).start()
        wr(j_last - 1, 0).wait()
        wr(j_last, 1).wait()

    return k


def kernel(x, pe):
    b, s = x.shape
    v, d = pe.shape
    n = b * s
    c = 32
    idx = x.reshape(_NW, (n // _NW) // c, c).astype(jnp.int32)
    out = _make_gather(n, v, d, c)(idx, pe)
    return out.reshape(b, s, d)


# speedup vs baseline: 2.3416x; 1.0198x over previous
import functools

import jax
import jax.numpy as jnp
from jax import lax
from jax.experimental import pallas as pl
from jax.experimental.pallas import tpu as pltpu
from jax.experimental.pallas import tpu_sc as plsc

_NC = 2
_NS = 16
_NW = _NC * _NS


@functools.lru_cache(maxsize=None)
def _make_gather(n, v, d, c):
    nbuf = 3
    n_per_w = n // _NW
    n_chunks = n_per_w // c
    assert n_chunks >= 8 and (n_chunks - 5) % nbuf == 0
    mesh = plsc.VectorSubcoreMesh(core_axis_name="c", subcore_axis_name="s")

    @functools.partial(
        pl.kernel,
        mesh=mesh,
        out_type=jax.ShapeDtypeStruct((n, d), jnp.float32),
        scratch_types=[
            pltpu.VMEM((n_chunks, c), jnp.int32),
            pltpu.VMEM((nbuf, c, d), jnp.float32),
            pltpu.SemaphoreType.DMA,
            pltpu.SemaphoreType.DMA,
            pltpu.SemaphoreType.DMA,
            pltpu.SemaphoreType.DMA,
            pltpu.SemaphoreType.DMA,
            pltpu.SemaphoreType.DMA,
        ],
    )
    def k(idx_hbm, table_hbm, out_hbm, idx_v, rows_v, g0, g1, g2, w0, w1, w2):
        wid = lax.axis_index("s") * _NC + lax.axis_index("c")
        base = wid * n_per_w
        gsems = (g0, g1, g2)
        wsems = (w0, w1, w2)
        pltpu.sync_copy(idx_hbm.at[wid], idx_v)

        def gath(j, b):
            return pltpu.make_async_copy(
                table_hbm.at[idx_v.at[j]], rows_v.at[b], gsems[b])

        def wr(j, b):
            return pltpu.make_async_copy(
                rows_v.at[b], out_hbm.at[pl.ds(base + j * c, c)], wsems[b])

        for j in range(nbuf):
            gath(j, j).start()
        gath(0, 0).wait()
        wr(0, 0).start()
        gath(1, 1).wait()
        wr(1, 1).start()
        wr(0, 0).wait()
        gath(3, 0).start()

        def step(j, b):
            gath(j, b).wait()
            wr(j, b).start()
            wr(j - 1, (b - 1) % nbuf).wait()
            gath(j + 2, (b - 1) % nbuf).start()

        def body(i, carry):
            for r in range(nbuf):
                step(nbuf * i + 2 + r, (2 + r) % nbuf)
            return carry

        lax.fori_loop(0, (n_chunks - 5) // nbuf, body, 0)
        jt = n_chunks - 3
        step(jt, jt % nbuf)
        gath(jt + 1, (jt + 1) % nbuf).wait()
        wr(jt + 1, (jt + 1) % nbuf).start()
        wr(jt, jt % nbuf).wait()
        gath(jt + 2, (jt + 2) % nbuf).wait()
        wr(jt + 2, (jt + 2) % nbuf).start()
        wr(jt + 1, (jt + 1) % nbuf).wait()
        wr(jt + 2, (jt + 2) % nbuf).wait()

    return k


def kernel(x, pe):
    b, s = x.shape
    v, d = pe.shape
    n = b * s
    c = 32
    idx = x.reshape(_NW, (n // _NW) // c, c).astype(jnp.int32)
    out = _make_gather(n, v, d, c)(idx, pe)
    return out.reshape(b, s, d)
